# baseline (speedup 1.0000x reference)
import functools

import jax
import jax.numpy as jnp
from jax import lax
from jax.experimental import pallas as pl
from jax.experimental.pallas import tpu as pltpu
from jax.experimental.pallas import tpu_sc as plsc

N = 50000
D = 256
S = 512
NC = 2
NS = 16
NW = NC * NS
SEG_PER_W = S // NW
R = 32
K = D // 16

_GDN = lax.GatherDimensionNumbers(
    offset_dims=(), collapsed_slice_dims=(0,), start_index_map=(0,))


def _lane_shuffle(v, idx):
    return lax.gather(v, idx[:, None], _GDN, slice_sizes=(1,),
                      mode=lax.GatherScatterMode.PROMISE_IN_BOUNDS)


@functools.partial(
    pl.kernel,
    out_type=jax.ShapeDtypeStruct((S, D), jnp.float32),
    mesh=plsc.VectorSubcoreMesh(core_axis_name="c", subcore_axis_name="s"),
    scratch_types=[
        pltpu.VMEM((R, D), jnp.float32),
        pltpu.VMEM((R, D), jnp.float32),
        pltpu.VMEM((D + 16,), jnp.float32),
        pltpu.VMEM((N + 16,), jnp.int32),
        pltpu.VMEM((SEG_PER_W, D), jnp.float32),
        pltpu.SemaphoreType.DMA,
        pltpu.SemaphoreType.DMA,
    ],
)
def _gatp_sc(x_hbm, b_hbm, par_hbm, out_hbm, xbuf0, xbuf1, par_v, bb, obuf,
             sem0, sem1):
    wid = lax.axis_index("s") * NC + lax.axis_index("c")
    seg0 = wid * SEG_PER_W

    pltpu.sync_copy(par_hbm, par_v)
    pltpu.sync_copy(b_hbm, bb.at[pl.ds(0, N)])

    w = [par_v[pl.ds(16 * j, 16)] for j in range(K)]
    bvec = par_v[pl.ds(D, 16)]
    lane = lax.iota(jnp.int32, 16)
    perms = [lane ^ c for c in (8, 4, 2, 1)]

    def lower_bound(s):
        def it(_, lohi):
            lo, hi = lohi
            mid = (lo + hi) // 2
            v = bb[pl.ds(mid, 16)][0]
            open_ = lo < hi
            lt = jnp.logical_and(open_, v < s)
            ge = jnp.logical_and(open_, jnp.logical_not(v < s))
            return (jnp.where(lt, mid + 1, lo), jnp.where(ge, mid, hi))
        lo, _ = lax.fori_loop(0, 16, it, (jnp.int32(0), jnp.int32(N)))
        return lo

    zero = jnp.zeros((16,), jnp.float32)

    def seg_body(sl, rs):
        re = lower_bound(seg0 + sl + 1)
        rs_a = (rs // 8) * 8
        nblk = lax.div(re - rs_a + (R - 1), R)

        def start_c(wi):
            return jnp.minimum(rs_a + wi * R, N - R)

        def issue(wi, buf, sem):
            pltpu.async_copy(x_hbm.at[pl.ds(start_c(wi), R)], buf, sem)

        def wait(wi, buf, sem):
            pltpu.make_async_copy(
                x_hbm.at[pl.ds(start_c(wi), R)], buf, sem).wait()

        def compute(wi, buf, carry):
            d = carry[0]
            accs = list(carry[1:])
            start_u = rs_a + wi * R
            start = start_c(wi)
            lo_b = jnp.maximum(rs, start_u)

            def row_body(r, c):
                d = c[0]
                accs = c[1:]
                row = start + r
                xr = [buf[r, pl.ds(16 * j, 16)] for j in range(K)]
                part = xr[0] * w[0]
                for j in range(1, K):
                    part = part + xr[j] * w[j]
                for p in perms:
                    part = part + _lane_shuffle(part, p)
                valid = jnp.logical_and(row >= lo_b, row < re)
                scale = jnp.where(valid, 1.0, 0.0)
                e = jnp.exp(part + bvec) * scale
                return (d + e,) + tuple(
                    a + e * xj for a, xj in zip(accs, xr))

            return lax.fori_loop(0, R, row_body, (d,) + tuple(accs),
                                 unroll=8)

        def blk(wi, carry):
            pltpu.sync_copy(x_hbm.at[pl.ds(start_c(wi), R)], xbuf0)
            return compute(wi, xbuf0, carry)

        res = lax.fori_loop(0, nblk, blk, (zero,) * (K + 1))
        inv = 1.0 / (res[0] + 1e-16)
        for j in range(K):
            obuf[sl, pl.ds(16 * j, 16)] = res[1 + j] * inv
        return re

    lax.fori_loop(0, SEG_PER_W, seg_body, lower_bound(seg0))
    pltpu.sync_copy(obuf, out_hbm.at[pl.ds(seg0, SEG_PER_W)])


def kernel(x, batch, W, b):
    b32 = batch.astype(jnp.int32)
    params = jnp.concatenate(
        [W.reshape(-1), jnp.broadcast_to(b.reshape(-1)[:1], (16,))]
    ).astype(jnp.float32)
    return _gatp_sc(x, b32, params)


# speedup vs baseline: 1.8088x; 1.8088x over previous
import functools

import jax
import jax.numpy as jnp
from jax import lax
from jax.experimental import pallas as pl
from jax.experimental.pallas import tpu as pltpu
from jax.experimental.pallas import tpu_sc as plsc

N = 50000
D = 256
S = 512
NC = 2
NS = 16
NW = NC * NS
SEG_PER_W = S // NW
R = 32
K = D // 16

_GDN = lax.GatherDimensionNumbers(
    offset_dims=(), collapsed_slice_dims=(0,), start_index_map=(0,))


def _lane_shuffle(v, idx):
    return lax.gather(v, idx[:, None], _GDN, slice_sizes=(1,),
                      mode=lax.GatherScatterMode.PROMISE_IN_BOUNDS)


@functools.partial(
    pl.kernel,
    out_type=jax.ShapeDtypeStruct((S, D), jnp.float32),
    mesh=plsc.VectorSubcoreMesh(core_axis_name="c", subcore_axis_name="s"),
    scratch_types=[
        pltpu.VMEM((2 * R, D), jnp.float32),
        pltpu.VMEM((D + 16,), jnp.float32),
        pltpu.VMEM((N + 16,), jnp.int32),
        pltpu.VMEM((SEG_PER_W, D), jnp.float32),
        pltpu.SemaphoreType.DMA,
    ],
)
def _gatp_sc(x_hbm, b_hbm, par_hbm, out_hbm, xbuf, par_v, bb, obuf, sem0):
    wid = lax.axis_index("s") * NC + lax.axis_index("c")
    seg0 = wid * SEG_PER_W

    pltpu.sync_copy(par_hbm, par_v)
    pltpu.sync_copy(b_hbm, bb.at[pl.ds(0, N)])

    w = [par_v[pl.ds(16 * j, 16)] for j in range(K)]
    bvec = par_v[pl.ds(D, 16)]
    lane = lax.iota(jnp.int32, 16)
    perms = [lane ^ c for c in (8, 4, 2, 1)]

    def lower_bound(s):
        def it(_, lohi):
            lo, hi = lohi
            mid = (lo + hi) // 2
            v = bb[pl.ds(mid, 16)][0]
            open_ = lo < hi
            lt = jnp.logical_and(open_, v < s)
            ge = jnp.logical_and(open_, jnp.logical_not(v < s))
            return (jnp.where(lt, mid + 1, lo), jnp.where(ge, mid, hi))
        lo, _ = lax.fori_loop(0, 16, it, (jnp.int32(0), jnp.int32(N)))
        return lo

    zero = jnp.zeros((16,), jnp.float32)

    def seg_body(sl, rs):
        re = lower_bound(seg0 + sl + 1)
        rs_a = (rs // 8) * 8
        nblk = lax.div(re - rs_a + (R - 1), R)

        def start_c(wi):
            return jnp.minimum(rs_a + wi * R, N - R)

        def pbase(wi):
            return (wi % 2) * R

        def issue(wi):
            pltpu.async_copy(x_hbm.at[pl.ds(start_c(wi), R)],
                             xbuf.at[pl.ds(pbase(wi), R)], sem0)

        def wait(wi):
            pltpu.make_async_copy(
                x_hbm.at[pl.ds(start_c(wi), R)],
                xbuf.at[pl.ds(pbase(wi), R)], sem0).wait()

        @pl.when(nblk > 0)
        def _():
            issue(0)

        def blk(wi, carry):
            d = carry[0]
            accs = list(carry[1:])
            start_u = rs_a + wi * R
            start = start_c(wi)
            pb = pbase(wi)
            lo_b = jnp.maximum(rs, start_u)
            wait(wi)

            @pl.when(wi + 1 < nblk)
            def _():
                issue(wi + 1)

            for r in range(R):
                row = start + r
                xr = [xbuf[pb + r, pl.ds(16 * j, 16)] for j in range(K)]
                part = xr[0] * w[0]
                for j in range(1, K):
                    part = part + xr[j] * w[j]
                for p in perms:
                    part = part + _lane_shuffle(part, p)
                valid = jnp.logical_and(row >= lo_b, row < re)
                scale = jnp.where(valid, 1.0, 0.0)
                e = jnp.exp(part + bvec) * scale
                d = d + e
                accs = [a + e * xj for a, xj in zip(accs, xr)]
            return (d,) + tuple(accs)

        res = lax.fori_loop(0, nblk, blk, (zero,) * (K + 1))
        inv = 1.0 / (res[0] + 1e-16)
        for j in range(K):
            obuf[sl, pl.ds(16 * j, 16)] = res[1 + j] * inv
        return re

    lax.fori_loop(0, SEG_PER_W, seg_body, lower_bound(seg0))
    pltpu.sync_copy(obuf, out_hbm.at[pl.ds(seg0, SEG_PER_W)])


def kernel(x, batch, W, b):
    b32 = batch.astype(jnp.int32)
    params = jnp.concatenate(
        [W.reshape(-1), jnp.broadcast_to(b.reshape(-1)[:1], (16,))]
    ).astype(jnp.float32)
    return _gatp_sc(x, b32, params)
